# baseline (speedup 1.0000x reference)
import functools

import jax
import jax.numpy as jnp
from jax import lax
from jax.experimental import pallas as pl
from jax.experimental.pallas import tpu as pltpu
from jax.experimental.pallas import tpu_sc as plsc

NUM_USERS = 1000000
NUM_MOVIES = 100000
NUM_GENRES = 32
NUM_YEARS = 120
YEARS_PAD = 128
D = 32
B = 16384

NC = 2
NS = 16
L = 16
NW = NC * NS
BPW = B // NW
CH = 128
NCH = BPW // CH
GROUPS = BPW // L

EPS = 1e-8
EPS2 = EPS * EPS


def _rsqrt(x):
    i = plsc.bitcast(x, jnp.int32)
    y = plsc.bitcast(jnp.int32(0x5F3759DF) - (i >> 1), jnp.float32)
    for _ in range(3):
        y = y * (1.5 - 0.5 * x * y * y)
    return y


def _inv_norm(n2):
    return jnp.where(n2 > EPS2, _rsqrt(n2), jnp.float32(1.0 / EPS))


def _normalize_table(tab, rows):
    iota = lax.iota(jnp.int32, L)

    def body(grp, _):
        rvec = grp * L + iota
        n2 = jnp.zeros((L,), jnp.float32)
        for d in range(D):
            dv = jnp.full((L,), d, jnp.int32)
            v = plsc.load_gather(tab, [rvec, dv])
            n2 = n2 + v * v
        inv = _inv_norm(n2)
        for d in range(D):
            dv = jnp.full((L,), d, jnp.int32)
            v = plsc.load_gather(tab, [rvec, dv])
            plsc.store_scatter(tab, [rvec, dv], v * inv)
        return _

    lax.fori_loop(0, rows // L, body, 0)


def _body(user_idx, movie_idx, genre_idx, year_idx,
          user_embeds, movie_embeds, user_biases, movie_biases,
          genre_embeds, year_embeds, out,
          uidx_v, midx_v, gidx_v, yidx_v,
          u_rows, m_rows, ub_v, mb_v, gtab, ytab, out_v, sem):
    wid = lax.axis_index("s") * NC + lax.axis_index("c")
    base = wid * BPW

    for j in range(NCH):
        pltpu.sync_copy(user_idx.at[pl.ds(base + j * CH, CH)], uidx_v.at[j])
        pltpu.sync_copy(movie_idx.at[pl.ds(base + j * CH, CH)], midx_v.at[j])
    pltpu.sync_copy(genre_idx.at[pl.ds(base, BPW)], gidx_v)
    pltpu.sync_copy(year_idx.at[pl.ds(base, BPW)], yidx_v)

    copies = []
    for j in range(NCH):
        sl = pl.ds(j * CH, CH)
        copies.append(pltpu.async_copy(
            user_embeds.at[uidx_v.at[j]], u_rows.at[sl], sem))
        copies.append(pltpu.async_copy(
            movie_embeds.at[midx_v.at[j]], m_rows.at[sl], sem))
        copies.append(pltpu.async_copy(
            user_biases.at[uidx_v.at[j]], ub_v.at[sl], sem))
        copies.append(pltpu.async_copy(
            movie_biases.at[midx_v.at[j]], mb_v.at[sl], sem))

    pltpu.sync_copy(genre_embeds, gtab)
    pltpu.sync_copy(year_embeds, ytab.at[pl.ds(0, NUM_YEARS)])
    zero = jnp.zeros((L,), jnp.float32)
    for r in range(NUM_YEARS, YEARS_PAD):
        ytab[r, pl.ds(0, L)] = zero
        ytab[r, pl.ds(L, L)] = zero
    _normalize_table(gtab, NUM_GENRES)
    _normalize_table(ytab, YEARS_PAD)

    for c in copies:
        c.wait()

    iota = lax.iota(jnp.int32, L)

    def group_body(g, _):
        e0 = g * L
        elem = e0 + iota
        gi = gidx_v[pl.ds(e0, L)]
        yi = yidx_v[pl.ds(e0, L)]
        d_um = jnp.zeros((L,), jnp.float32)
        d_ug = jnp.zeros((L,), jnp.float32)
        d_uy = jnp.zeros((L,), jnp.float32)
        n_u = jnp.zeros((L,), jnp.float32)
        n_m = jnp.zeros((L,), jnp.float32)
        for d in range(D):
            dv = jnp.full((L,), d, jnp.int32)
            u = plsc.load_gather(u_rows, [elem, dv])
            m = plsc.load_gather(m_rows, [elem, dv])
            gv = plsc.load_gather(gtab, [gi, dv])
            yv = plsc.load_gather(ytab, [yi, dv])
            d_um = d_um + u * m
            d_ug = d_ug + u * gv
            d_uy = d_uy + u * yv
            n_u = n_u + u * u
            n_m = n_m + m * m
        inv_u = _inv_norm(n_u)
        inv_m = _inv_norm(n_m)
        ub = ub_v[pl.ds(e0, L)]
        mb = mb_v[pl.ds(e0, L)]
        res = (d_um * inv_u * inv_m * 2.5 + 2.75
               + ub + mb + (d_ug + d_uy) * inv_u)
        out_v[pl.ds(e0, L)] = res
        return _

    lax.fori_loop(0, GROUPS, group_body, 0)

    pltpu.sync_copy(out_v, out.at[pl.ds(base, BPW)])


_sc_call = pl.kernel(
    _body,
    out_type=jax.ShapeDtypeStruct((B,), jnp.float32),
    mesh=plsc.VectorSubcoreMesh(
        core_axis_name="c", subcore_axis_name="s",
        num_cores=NC, num_subcores=NS),
    scratch_types=[
        pltpu.VMEM((NCH, CH), jnp.int32),
        pltpu.VMEM((NCH, CH), jnp.int32),
        pltpu.VMEM((BPW,), jnp.int32),
        pltpu.VMEM((BPW,), jnp.int32),
        pltpu.VMEM((BPW, D), jnp.float32),
        pltpu.VMEM((BPW, D), jnp.float32),
        pltpu.VMEM((BPW,), jnp.float32),
        pltpu.VMEM((BPW,), jnp.float32),
        pltpu.VMEM((NUM_GENRES, D), jnp.float32),
        pltpu.VMEM((YEARS_PAD, D), jnp.float32),
        pltpu.VMEM((BPW,), jnp.float32),
        pltpu.SemaphoreType.DMA,
    ],
    compiler_params=pltpu.CompilerParams(
        needs_layout_passes=False, use_tc_tiling_on_sc=False),
    name="mf_side_features_bias_sc",
)


@jax.jit
def kernel(user_idx, movie_idx, genre_idx, year_idx,
           user_embeds, movie_embeds, user_biases, movie_biases,
           genre_embeds, year_embeds):
    return _sc_call(
        user_idx.astype(jnp.int32), movie_idx.astype(jnp.int32),
        genre_idx.astype(jnp.int32), year_idx.astype(jnp.int32),
        user_embeds, movie_embeds,
        user_biases.reshape(-1), movie_biases.reshape(-1),
        genre_embeds, year_embeds)
